# baseline (speedup 1.0000x reference)
import functools

import jax
import jax.numpy as jnp
from jax.experimental import pallas as pl
from jax.experimental.pallas import tpu as pltpu

_G = 64


def _pick_block(n):
    for r in (1280, 640, 320):
        if n % r == 0:
            return r
    return n


def _fused_kernel(meta_ref, off_ref, batch_ref, glab_ref, x_ref,
                  w_enc_ref, b_enc_ref, w1_ref, b1_ref, gamma1_ref,
                  beta1_ref, w2_ref, b2_ref, out_ref, acc_ref, emb_ref,
                  *, nb, r, s_total):
    i = pl.program_id(0)
    ng = r // _G
    nf = ng * 8

    @pl.when(i == 0)
    def _init():
        acc_ref[...] = jnp.full_like(acc_ref, -jnp.inf)

    emb = jnp.dot(x_ref[...], w_enc_ref[...],
                  preferred_element_type=jnp.float32)
    emb_ref[...] = emb

    folded = emb.reshape(ng, 8, 8, emb.shape[-1]).max(axis=1)
    folded = folded.reshape(nf, emb.shape[-1])
    glab = glab_ref[0]

    b_first = meta_ref[2 * i]
    b_last = meta_ref[2 * i + 1]
    base = i * r

    def body(k, _):
        s = b_first + k
        red = jnp.max(jnp.where(glab == s, folded, -jnp.inf),
                      axis=0, keepdims=True)
        a = jnp.clip(off_ref[s] - base, 0, r - 1)
        e1 = jnp.clip(off_ref[s + 1] - base - 1, 0, r - 1)

        def edge(row):
            st = (row // _G) * _G
            rows = emb_ref[pl.ds(st, _G), :]
            ss = batch_ref[0, pl.ds(st, _G), :]
            return jnp.max(jnp.where(ss == s, rows, -jnp.inf),
                           axis=0, keepdims=True)

        red = jnp.maximum(red, jnp.maximum(edge(a), edge(e1)))
        acc_ref[pl.ds(s, 1), :] = jnp.maximum(acc_ref[pl.ds(s, 1), :], red)
        return 0

    jax.lax.fori_loop(0, b_last - b_first + 1, body, 0)

    @pl.when(i == nb - 1)
    def _final():
        g = acc_ref[...] + b_enc_ref[...]
        h = jnp.dot(g, w1_ref[...],
                    preferred_element_type=jnp.float32) + b1_ref[...]
        mu = jnp.mean(h, axis=-1, keepdims=True)
        var = jnp.mean((h - mu) * (h - mu), axis=-1, keepdims=True)
        h = (h - mu) * jax.lax.rsqrt(var + 1e-5) * gamma1_ref[...] \
            + beta1_ref[...]
        h = jnp.maximum(h, 0.0)
        out_ref[...] = jnp.dot(h, w2_ref[...],
                               preferred_element_type=jnp.float32) \
            + b2_ref[...]


def kernel(x, batch, W_enc, b_enc, W1, b1, gamma1, beta1, W2, b2):
    n, d = x.shape
    h = W1.shape[1]
    nt = W2.shape[1]
    s_total = 512
    r = _pick_block(n)
    nb = n // r
    ng = r // _G
    nf = ng * 8

    batch = batch.astype(jnp.int32)
    batch3 = batch.reshape(nb, r, 1)
    blk = batch.reshape(nb, r)
    meta = jnp.stack([blk[:, 0], blk[:, -1]], axis=1).reshape(-1)
    off = jnp.searchsorted(batch, jnp.arange(s_total + 1, dtype=jnp.int32),
                           side="left").astype(jnp.int32)
    grp = batch.reshape(n // _G, _G)
    lab = jnp.where(grp[:, 0] == grp[:, -1], grp[:, 0], -1)
    glab = jnp.repeat(lab, 8).reshape(nb, nf, 1)

    grid_spec = pltpu.PrefetchScalarGridSpec(
        num_scalar_prefetch=2,
        grid=(nb,),
        in_specs=[
            pl.BlockSpec((1, r, 1), lambda i, m, o: (i, 0, 0)),
            pl.BlockSpec((1, nf, 1), lambda i, m, o: (i, 0, 0)),
            pl.BlockSpec((r, d), lambda i, m, o: (i, 0)),
            pl.BlockSpec((d, d), lambda i, m, o: (0, 0)),
            pl.BlockSpec((1, d), lambda i, m, o: (0, 0)),
            pl.BlockSpec((d, h), lambda i, m, o: (0, 0)),
            pl.BlockSpec((1, h), lambda i, m, o: (0, 0)),
            pl.BlockSpec((1, h), lambda i, m, o: (0, 0)),
            pl.BlockSpec((1, h), lambda i, m, o: (0, 0)),
            pl.BlockSpec((h, nt), lambda i, m, o: (0, 0)),
            pl.BlockSpec((1, nt), lambda i, m, o: (0, 0)),
        ],
        out_specs=pl.BlockSpec((s_total, nt), lambda i, m, o: (0, 0)),
        scratch_shapes=[pltpu.VMEM((s_total, d), jnp.float32),
                        pltpu.VMEM((r, d), jnp.float32)],
    )

    fn = functools.partial(_fused_kernel, nb=nb, r=r, s_total=s_total)
    out = pl.pallas_call(
        fn,
        grid_spec=grid_spec,
        out_shape=jax.ShapeDtypeStruct((s_total, nt), jnp.float32),
    )(meta, off, batch3, glab, x, W_enc, b_enc.reshape(1, d),
      W1, b1.reshape(1, h), gamma1.reshape(1, h), beta1.reshape(1, h),
      W2, b2.reshape(1, nt))
    return out


# speedup vs baseline: 1.4387x; 1.4387x over previous
import functools

import jax
import jax.numpy as jnp
from jax.experimental import pallas as pl
from jax.experimental.pallas import tpu as pltpu


def _pick_block(n):
    for r in (1280, 640, 320, 160, 80, 40, 16, 8):
        if n % r == 0:
            return r
    return n


def _fused_kernel(meta_ref, batch_ref, x_ref, w_enc_ref, b_enc_ref,
                  w1_ref, b1_ref, gamma1_ref, beta1_ref, w2_ref, b2_ref,
                  out_ref, acc_ref, *, nb, s_total):
    i = pl.program_id(0)

    @pl.when(i == 0)
    def _init():
        acc_ref[...] = jnp.full_like(acc_ref, -jnp.inf)

    emb = jnp.dot(x_ref[...], w_enc_ref[...],
                  preferred_element_type=jnp.float32)
    seg = batch_ref[0]
    b_first = meta_ref[2 * i]
    b_last = meta_ref[2 * i + 1]

    def body(k, _):
        s = b_first + k
        red = jnp.max(jnp.where(seg == s, emb, -jnp.inf),
                      axis=0, keepdims=True)
        acc_ref[pl.ds(s, 1), :] = jnp.maximum(acc_ref[pl.ds(s, 1), :], red)
        return 0

    jax.lax.fori_loop(0, b_last - b_first + 1, body, 0)

    @pl.when(i == nb - 1)
    def _final():
        g = acc_ref[...] + b_enc_ref[...]
        h = jnp.dot(g, w1_ref[...],
                    preferred_element_type=jnp.float32) + b1_ref[...]
        mu = jnp.mean(h, axis=-1, keepdims=True)
        var = jnp.mean((h - mu) * (h - mu), axis=-1, keepdims=True)
        h = (h - mu) * jax.lax.rsqrt(var + 1e-5) * gamma1_ref[...] \
            + beta1_ref[...]
        h = jnp.maximum(h, 0.0)
        out_ref[...] = jnp.dot(h, w2_ref[...],
                               preferred_element_type=jnp.float32) \
            + b2_ref[...]


def kernel(x, batch, W_enc, b_enc, W1, b1, gamma1, beta1, W2, b2):
    n, d = x.shape
    h = W1.shape[1]
    nt = W2.shape[1]
    s_total = 512
    r = _pick_block(n)
    nb = n // r

    batch = batch.astype(jnp.int32)
    batch3 = batch.reshape(nb, r, 1)
    blk = batch.reshape(nb, r)
    meta = jnp.stack([blk[:, 0], blk[:, -1]], axis=1).reshape(-1)

    grid_spec = pltpu.PrefetchScalarGridSpec(
        num_scalar_prefetch=1,
        grid=(nb,),
        in_specs=[
            pl.BlockSpec((1, r, 1), lambda i, meta: (i, 0, 0)),
            pl.BlockSpec((r, d), lambda i, meta: (i, 0)),
            pl.BlockSpec((d, d), lambda i, meta: (0, 0)),
            pl.BlockSpec((1, d), lambda i, meta: (0, 0)),
            pl.BlockSpec((d, h), lambda i, meta: (0, 0)),
            pl.BlockSpec((1, h), lambda i, meta: (0, 0)),
            pl.BlockSpec((1, h), lambda i, meta: (0, 0)),
            pl.BlockSpec((1, h), lambda i, meta: (0, 0)),
            pl.BlockSpec((h, nt), lambda i, meta: (0, 0)),
            pl.BlockSpec((1, nt), lambda i, meta: (0, 0)),
        ],
        out_specs=pl.BlockSpec((s_total, nt), lambda i, meta: (0, 0)),
        scratch_shapes=[pltpu.VMEM((s_total, d), jnp.float32)],
    )

    fn = functools.partial(_fused_kernel, nb=nb, s_total=s_total)
    out = pl.pallas_call(
        fn,
        grid_spec=grid_spec,
        out_shape=jax.ShapeDtypeStruct((s_total, nt), jnp.float32),
    )(meta, batch3, x, W_enc, b_enc.reshape(1, d),
      W1, b1.reshape(1, h), gamma1.reshape(1, h), beta1.reshape(1, h),
      W2, b2.reshape(1, nt))
    return out
